# Initial kernel scaffold; baseline (speedup 1.0000x reference)
import jax
import jax.numpy as jnp
from jax.experimental import pallas as pl


def kernel(x):
    raise NotImplementedError("write your pallas kernel here")


# speedup vs baseline: 1.3737x; 1.3737x over previous
import jax
import jax.numpy as jnp
from jax.experimental import pallas as pl
from jax.experimental.pallas import tpu as pltpu

_K = 100
_CAP = 128
_THRESH = 0.25

_NT = (((1,), (1,)), ((), ()))


def _body(x_ref, out_ref):
    xb = x_ref[0]
    ncls = xb.shape[0] - 4
    n = xb.shape[1]
    cx = xb[0:1, :]
    cy = xb[1:2, :]
    hw = xb[2:3, :] * 0.5
    hh = xb[3:4, :] * 0.5
    scores = xb[4:, :]

    best = jnp.max(scores, axis=0, keepdims=True)
    ci = jax.lax.broadcasted_iota(jnp.int32, (ncls, n), 0)
    cls = jnp.min(jnp.where(scores == best, ci, ncls * 2),
                  axis=0, keepdims=True)

    bits = jax.lax.bitcast_convert_type(best, jnp.int32)
    key = jnp.where(bits >= 0, bits, bits ^ jnp.int32(0x7FFFFFFF))

    def count_ge(t):
        return jnp.sum((key >= t).astype(jnp.int32))

    min32 = jnp.int32(-2147483647 - 1)
    t0 = jnp.where(count_ge(jnp.int32(0)) >= _K, jnp.int32(0), min32)

    def bs_body(i, t):
        cand = t | (jnp.int32(1) << (30 - i))
        return jnp.where(count_ge(cand) >= _K, cand, t)

    thr = jax.lax.fori_loop(0, 31, bs_body, t0)

    mask = key >= thr
    cnt = jnp.minimum(jnp.sum(mask.astype(jnp.int32)), _CAP)

    lane = jax.lax.broadcasted_iota(jnp.int32, (1, n), 1)
    m = mask.astype(jnp.int32)
    d = 1
    while d < n:
        m = m + jnp.where(lane >= d, pltpu.roll(m, d, axis=1), 0)
        d *= 2
    pos = jnp.where(mask, m - 1, jnp.int32(-5))

    idxf = lane.astype(jnp.float32)
    vals8 = jnp.concatenate(
        [best, idxf, cls.astype(jnp.float32),
         cx - hw, cy - hh, cx + hw, cy + hh,
         jnp.zeros_like(best)], axis=0)

    pos_c = jnp.transpose(pos)
    p2 = (pos_c == jax.lax.broadcasted_iota(jnp.int32, (n, _CAP), 1))
    acc = jnp.dot(vals8, p2.astype(jnp.float32),
                  preferred_element_type=jnp.float32,
                  precision=jax.lax.Precision.HIGHEST)

    s = acc[0:1, :]
    ii = acc[1:2, :]
    clane = jax.lax.broadcasted_iota(jnp.int32, (1, _CAP), 1)
    valid = clane < cnt
    s_r = jnp.broadcast_to(s, (_CAP, _CAP))
    i_r = jnp.broadcast_to(ii, (_CAP, _CAP))
    v_r = jnp.broadcast_to(valid, (_CAP, _CAP))
    s_c = jnp.transpose(s_r)
    i_c = jnp.transpose(i_r)
    v_c = jnp.transpose(v_r)
    beats = ((s_c > s_r) | ((s_c == s_r) & (i_c < i_r))) & v_c
    rank = jnp.sum(beats.astype(jnp.int32), axis=0, keepdims=True)
    finalpos = jnp.where(valid, rank, _CAP - 1)

    fp_c = jnp.transpose(finalpos)
    q2 = (fp_c == jax.lax.broadcasted_iota(jnp.int32, (_CAP, _CAP), 1))
    sorted8 = jnp.dot(acc, q2.astype(jnp.float32),
                      preferred_element_type=jnp.float32,
                  precision=jax.lax.Precision.HIGHEST)

    ndet = jnp.sum(((s > _THRESH) & valid & (rank < _K)).astype(jnp.int32))
    nrow = jnp.full((1, _CAP), ndet.astype(jnp.float32))
    out_ref[0] = jnp.concatenate([sorted8[0:7, :], nrow], axis=0)


def kernel(x):
    b, c, n = x.shape
    r = pl.pallas_call(
        _body,
        grid=(b,),
        in_specs=[pl.BlockSpec((1, c, n), lambda i: (i, 0, 0))],
        out_specs=pl.BlockSpec((1, 8, _CAP), lambda i: (i, 0, 0)),
        out_shape=jax.ShapeDtypeStruct((b, 8, _CAP), jnp.float32),
    )(x)
    det_scores = r[:, 0, :_K]
    det_classes = r[:, 2, :_K].astype(jnp.int32)
    det_boxes = jnp.transpose(r[:, 3:7, :_K], (0, 2, 1))
    num_det = r[:, 7, :1].astype(jnp.int32)
    return (num_det, det_boxes, det_scores, det_classes)
